# Initial kernel scaffold; baseline (speedup 1.0000x reference)
import functools

import jax
import jax.numpy as jnp
from jax import lax
from jax.experimental import pallas as pl
from jax.experimental.pallas import tpu as pltpu
from jax.experimental.pallas import tpu_sc as plsc

B, L = 16384, 50
N = B * L
ASN_D = 19
GEO_V, GEO_D = 252, 8
OUT_D = 32

NC, NS = 2, 16
NW = NC * NS
PER_W = N // NW
T = 1024
NCHUNK = PER_W // T
G = T // 16
IDX_ROWS = T // 128


def _body(asn_table, geo_table, ips_table, asn_idx, geo_idx, ips_idx,
          lat, lon, out,
          asn_idx_v, geo_idx_v, ips_idx_v, lat_v, lon_v,
          asn_rows_v, geo_tab_v, ips_tab_v, outb, sem):
    wid = lax.axis_index("s") * NC + lax.axis_index("c")
    wbase = wid * PER_W

    pltpu.sync_copy(geo_table, geo_tab_v)
    pltpu.sync_copy(ips_table, ips_tab_v)

    iota = lax.iota(jnp.int32, 16)
    dst_base = iota * OUT_D

    def chunk_body(i, carry):
        base = wbase + i * T
        pltpu.sync_copy(asn_idx.at[pl.ds(base // 128, IDX_ROWS)], asn_idx_v)
        pltpu.sync_copy(geo_idx.at[pl.ds(base, T)], geo_idx_v)
        pltpu.sync_copy(ips_idx.at[pl.ds(base, T)], ips_idx_v)
        pltpu.sync_copy(lat.at[pl.ds(base, T)], lat_v)
        pltpu.sync_copy(lon.at[pl.ds(base, T)], lon_v)
        copies = []
        for j in range(IDX_ROWS):
            copies.append(pltpu.async_copy(
                asn_table.at[asn_idx_v.at[j]],
                asn_rows_v.at[pl.ds(j * 128, 128)], sem))
        for c in copies:
            c.wait()

        def group_body(g, carry2):
            t0 = g * 16
            dst0 = dst_base + g * (16 * OUT_D)
            rows = iota + t0
            plsc.store_scatter(outb, [dst0], lat_v[pl.ds(t0, 16)])
            plsc.store_scatter(outb, [dst0 + 1], lon_v[pl.ds(t0, 16)])
            for c in range(ASN_D):
                v = plsc.load_gather(asn_rows_v,
                                     [rows, jnp.full((16,), c, jnp.int32)])
                plsc.store_scatter(outb, [dst0 + (2 + c)], v)
            gi = geo_idx_v[pl.ds(t0, 16)]
            for c in range(GEO_D):
                v = plsc.load_gather(geo_tab_v,
                                     [gi, jnp.full((16,), c, jnp.int32)])
                plsc.store_scatter(outb, [dst0 + (21 + c)], v)
            pi = ips_idx_v[pl.ds(t0, 16)]
            for c in range(3):
                v = plsc.load_gather(ips_tab_v,
                                     [pi, jnp.full((16,), c, jnp.int32)])
                plsc.store_scatter(outb, [dst0 + (29 + c)], v)
            return carry2

        lax.fori_loop(0, G, group_body, 0)
        pltpu.sync_copy(outb, out.at[pl.ds(base * OUT_D, T * OUT_D)])
        return carry

    lax.fori_loop(0, NCHUNK, chunk_body, 0)


@jax.jit
def _run(asn_table, geo_table, ips_table, asn_idx2d, geo_idx, ips_idx,
         lat, lon):
    mesh = plsc.VectorSubcoreMesh(core_axis_name="c", subcore_axis_name="s")
    return pl.kernel(
        _body,
        out_type=jax.ShapeDtypeStruct((N * OUT_D,), jnp.float32),
        mesh=mesh,
        scratch_types=[
            pltpu.VMEM((IDX_ROWS, 128), jnp.int32),
            pltpu.VMEM((T,), jnp.int32),
            pltpu.VMEM((T,), jnp.int32),
            pltpu.VMEM((T,), jnp.float32),
            pltpu.VMEM((T,), jnp.float32),
            pltpu.VMEM((T, ASN_D), jnp.float32),
            pltpu.VMEM((GEO_V, GEO_D), jnp.float32),
            pltpu.VMEM((8, 4), jnp.float32),
            pltpu.VMEM((T * OUT_D,), jnp.float32),
            pltpu.SemaphoreType.DMA,
        ],
    )(asn_table, geo_table, ips_table, asn_idx2d, geo_idx, ips_idx, lat, lon)


def kernel(x_lat, x_long, x_asn, x_geo_cc, x_ip_source,
           asn_table, geo_cc_table, ip_source_table):
    asn_idx2d = x_asn.reshape(N // 128, 128).astype(jnp.int32)
    geo_idx = x_geo_cc.reshape(N).astype(jnp.int32)
    ips_idx = x_ip_source.reshape(N).astype(jnp.int32)
    lat = x_lat.reshape(N)
    lon = x_long.reshape(N)
    ips_pad = jnp.pad(ip_source_table, ((0, 3), (0, 1)))
    out = _run(asn_table, geo_cc_table, ips_pad, asn_idx2d, geo_idx,
               ips_idx, lat, lon)
    return out.reshape(B, L, OUT_D)


# speedup vs baseline: 15.9355x; 15.9355x over previous
import functools

import jax
import jax.numpy as jnp
from jax import lax
from jax.experimental import pallas as pl
from jax.experimental.pallas import tpu as pltpu
from jax.experimental.pallas import tpu_sc as plsc

B, L = 16384, 50
N = B * L
ASN_D = 19
GEO_V, GEO_D = 252, 8
OUT_D = 32

NC, NS = 2, 16
NW = NC * NS
PER_W = N // NW
T = 1024
NCHUNK = PER_W // T
G = T // 16
IDX_ROWS = T // 128


def _body(asn_table, geo_table, ips_table, asn_idx, geo_idx, ips_idx,
          lat, lon, out,
          asn_idx_v, geo_idx_v, ips_idx_v, lat_v, lon_v,
          asn_rows_v, geo_tab_v, ips_tab_v, outb, sem):
    wid = lax.axis_index("s") * NC + lax.axis_index("c")
    wbase = wid * PER_W

    pltpu.sync_copy(geo_table, geo_tab_v)
    pltpu.sync_copy(ips_table, ips_tab_v)

    iota = lax.iota(jnp.int32, 16)
    dst_base = iota * OUT_D

    def chunk_body(i, carry):
        base = wbase + i * T
        row0 = pl.multiple_of(base // 128, 8)
        pltpu.sync_copy(asn_idx.at[pl.ds(row0, IDX_ROWS)], asn_idx_v)
        pltpu.sync_copy(geo_idx.at[pl.ds(base, T)], geo_idx_v)
        pltpu.sync_copy(ips_idx.at[pl.ds(base, T)], ips_idx_v)
        pltpu.sync_copy(lat.at[pl.ds(base, T)], lat_v)
        pltpu.sync_copy(lon.at[pl.ds(base, T)], lon_v)
        copies = []
        for j in range(IDX_ROWS):
            copies.append(pltpu.async_copy(
                asn_table.at[asn_idx_v.at[j]],
                asn_rows_v.at[pl.ds(j * 128, 128)], sem))
        for c in copies:
            c.wait()

        def group_body(g, carry2):
            t0 = g * 16
            dst0 = dst_base + g * (16 * OUT_D)
            rows = iota + t0
            plsc.store_scatter(outb, [dst0], lat_v[pl.ds(t0, 16)])
            plsc.store_scatter(outb, [dst0 + 1], lon_v[pl.ds(t0, 16)])
            for c in range(ASN_D):
                v = plsc.load_gather(asn_rows_v,
                                     [rows, jnp.full((16,), c, jnp.int32)])
                plsc.store_scatter(outb, [dst0 + (2 + c)], v)
            gi = geo_idx_v[pl.ds(t0, 16)]
            for c in range(GEO_D):
                v = plsc.load_gather(geo_tab_v,
                                     [gi, jnp.full((16,), c, jnp.int32)])
                plsc.store_scatter(outb, [dst0 + (21 + c)], v)
            pi = ips_idx_v[pl.ds(t0, 16)]
            for c in range(3):
                v = plsc.load_gather(ips_tab_v,
                                     [pi, jnp.full((16,), c, jnp.int32)])
                plsc.store_scatter(outb, [dst0 + (29 + c)], v)
            return carry2

        lax.fori_loop(0, G, group_body, 0)
        pltpu.sync_copy(outb, out.at[pl.ds(base * OUT_D, T * OUT_D)])
        return carry

    lax.fori_loop(0, NCHUNK, chunk_body, 0)


@jax.jit
def _run(asn_table, geo_table, ips_table, asn_idx2d, geo_idx, ips_idx,
         lat, lon):
    mesh = plsc.VectorSubcoreMesh(core_axis_name="c", subcore_axis_name="s")
    return pl.kernel(
        _body,
        out_type=jax.ShapeDtypeStruct((N * OUT_D,), jnp.float32),
        mesh=mesh,
        compiler_params=pltpu.CompilerParams(
            needs_layout_passes=False, use_tc_tiling_on_sc=False),
        scratch_types=[
            pltpu.VMEM((IDX_ROWS, 128), jnp.int32),
            pltpu.VMEM((T,), jnp.int32),
            pltpu.VMEM((T,), jnp.int32),
            pltpu.VMEM((T,), jnp.float32),
            pltpu.VMEM((T,), jnp.float32),
            pltpu.VMEM((T, ASN_D), jnp.float32),
            pltpu.VMEM((GEO_V, GEO_D), jnp.float32),
            pltpu.VMEM((8, 4), jnp.float32),
            pltpu.VMEM((T * OUT_D,), jnp.float32),
            pltpu.SemaphoreType.DMA,
        ],
    )(asn_table, geo_table, ips_table, asn_idx2d, geo_idx, ips_idx, lat, lon)


def kernel(x_lat, x_long, x_asn, x_geo_cc, x_ip_source,
           asn_table, geo_cc_table, ip_source_table):
    asn_idx2d = x_asn.reshape(N // 128, 128).astype(jnp.int32)
    geo_idx = x_geo_cc.reshape(N).astype(jnp.int32)
    ips_idx = x_ip_source.reshape(N).astype(jnp.int32)
    lat = x_lat.reshape(N)
    lon = x_long.reshape(N)
    ips_pad = jnp.pad(ip_source_table, ((0, 3), (0, 1)))
    out = _run(asn_table, geo_cc_table, ips_pad, asn_idx2d, geo_idx,
               ips_idx, lat, lon)
    return out.reshape(B, L, OUT_D)
